# baseline (speedup 1.0000x reference)
import functools

import jax
import jax.numpy as jnp
from jax import lax
from jax.experimental import pallas as pl
from jax.experimental.pallas import tpu as pltpu
from jax.experimental.pallas import tpu_sc as plsc

VOCAB = 1000000
EMBED_DIM = 16
PADDING_IDX = 0
BATCH = 4096
SEQ_LEN = 200
TOTAL = BATCH * SEQ_LEN

_INFO = plsc.get_sparse_core_info()
NC = _INFO.num_cores
NS = _INFO.num_subcores
NW = NC * NS
PER_W = TOTAL // NW
CHUNK = 2560
N_CHUNKS = PER_W // CHUNK


def _gather_body(tokens_hbm, table_hbm, out_hbm, idx_v, rows_v, sem):
    wid = lax.axis_index("s") * NC + lax.axis_index("c")
    base = wid * PER_W

    def step(i, _):
        off = base + i * CHUNK
        pltpu.sync_copy(tokens_hbm.at[pl.ds(off, CHUNK)], idx_v)
        pltpu.async_copy(table_hbm.at[idx_v], rows_v, sem).wait()
        pltpu.sync_copy(rows_v, out_hbm.at[pl.ds(off, CHUNK)])
        return 0

    lax.fori_loop(0, N_CHUNKS, step, 0, unroll=False)


_gather = functools.partial(
    pl.kernel,
    out_type=jax.ShapeDtypeStruct((TOTAL, EMBED_DIM), jnp.float32),
    mesh=plsc.VectorSubcoreMesh(core_axis_name="c", subcore_axis_name="s"),
    scratch_types=[
        pltpu.VMEM((CHUNK,), jnp.int32),
        pltpu.VMEM((CHUNK, EMBED_DIM), jnp.float32),
        pltpu.SemaphoreType.DMA,
    ],
    compiler_params=pltpu.CompilerParams(use_tc_tiling_on_sc=False),
)(_gather_body)


def _mask_body(tokens_ref, pad_ref, seq_ref):
    pad_ref[...] = tokens_ref[...] == PADDING_IDX
    row = lax.broadcasted_iota(jnp.int32, (SEQ_LEN, SEQ_LEN), 0)
    col = lax.broadcasted_iota(jnp.int32, (SEQ_LEN, SEQ_LEN), 1)
    seq_ref[...] = col > row


_masks = pl.pallas_call(
    _mask_body,
    out_shape=(
        jax.ShapeDtypeStruct((BATCH, SEQ_LEN), jnp.bool_),
        jax.ShapeDtypeStruct((SEQ_LEN, SEQ_LEN), jnp.bool_),
    ),
)


def kernel(tokens, table):
    flat = tokens.reshape(TOTAL)
    rows = _gather(flat, table)
    features = rows.reshape(BATCH, SEQ_LEN, EMBED_DIM)
    pad, seqm = _masks(tokens)
    return (features, (pad, seqm))


# speedup vs baseline: 1.0146x; 1.0146x over previous
import functools

import jax
import jax.numpy as jnp
from jax import lax
from jax.experimental import pallas as pl
from jax.experimental.pallas import tpu as pltpu
from jax.experimental.pallas import tpu_sc as plsc

VOCAB = 1000000
EMBED_DIM = 16
PADDING_IDX = 0
BATCH = 4096
SEQ_LEN = 200
TOTAL = BATCH * SEQ_LEN

_INFO = plsc.get_sparse_core_info()
NC = _INFO.num_cores
NS = _INFO.num_subcores
NW = NC * NS
PER_W = TOTAL // NW
CHUNK = 2560
N_CHUNKS = PER_W // CHUNK


def _gather_body(tokens_hbm, table_hbm, out_hbm, idx_v, rows_v,
                 gsem0, gsem1, wsem0, wsem1):
    wid = lax.axis_index("s") * NC + lax.axis_index("c")
    base = wid * PER_W
    gsems = (gsem0, gsem1)
    wsems = (wsem0, wsem1)

    pltpu.sync_copy(tokens_hbm.at[pl.ds(base, PER_W)], idx_v)

    def start_gather(i, b):
        return pltpu.async_copy(
            table_hbm.at[idx_v.at[pl.ds(i * CHUNK, CHUNK)]],
            rows_v.at[b], gsems[b])

    def start_write(i, b):
        return pltpu.async_copy(
            rows_v.at[b], out_hbm.at[pl.ds(base + i * CHUNK, CHUNK)],
            wsems[b])

    g = [None, None]
    w = [None, None]
    g[0] = start_gather(0, 0)
    for i in range(N_CHUNKS):
        b = i & 1
        if i + 1 < N_CHUNKS:
            if w[1 - b] is not None:
                w[1 - b].wait()
            g[1 - b] = start_gather(i + 1, 1 - b)
        g[b].wait()
        w[b] = start_write(i, b)
    w[0].wait()
    w[1].wait()


_gather = functools.partial(
    pl.kernel,
    out_type=jax.ShapeDtypeStruct((TOTAL, EMBED_DIM), jnp.float32),
    mesh=plsc.VectorSubcoreMesh(core_axis_name="c", subcore_axis_name="s"),
    scratch_types=[
        pltpu.VMEM((PER_W,), jnp.int32),
        pltpu.VMEM((2, CHUNK, EMBED_DIM), jnp.float32),
        pltpu.SemaphoreType.DMA,
        pltpu.SemaphoreType.DMA,
        pltpu.SemaphoreType.DMA,
        pltpu.SemaphoreType.DMA,
    ],
    compiler_params=pltpu.CompilerParams(use_tc_tiling_on_sc=False),
)(_gather_body)


def _mask_body(tokens_ref, pad_ref, seq_ref):
    pad_ref[...] = tokens_ref[...] == PADDING_IDX
    row = lax.broadcasted_iota(jnp.int32, (SEQ_LEN, SEQ_LEN), 0)
    col = lax.broadcasted_iota(jnp.int32, (SEQ_LEN, SEQ_LEN), 1)
    seq_ref[...] = col > row


_masks = pl.pallas_call(
    _mask_body,
    out_shape=(
        jax.ShapeDtypeStruct((BATCH, SEQ_LEN), jnp.bool_),
        jax.ShapeDtypeStruct((SEQ_LEN, SEQ_LEN), jnp.bool_),
    ),
)


def kernel(tokens, table):
    flat = tokens.reshape(TOTAL)
    rows = _gather(flat, table)
    features = rows.reshape(BATCH, SEQ_LEN, EMBED_DIM)
    pad, seqm = _masks(tokens)
    return (features, (pad, seqm))
